# baseline (speedup 1.0000x reference)
import functools

import jax
import jax.numpy as jnp
from jax import lax
from jax.experimental import pallas as pl
from jax.experimental.pallas import tpu as pltpu
from jax.experimental.pallas import tpu_sc as plsc

N = 100000
D = 128
G = 16
TOPK = 64
BLK = 20000
NBLK = N // BLK

NW = 32
SHARD = 3136
NCHUNK = SHARD // 16
POOL = 8


def _pass1_body(seg_ref, x_ref, gwt_ref, gb_ref, w1_ref, b1_ref, w2_ref,
                b2_ref, gate_out_ref, out_ref, c_out_ref,
                m_s, l_s, acc_s):
    i = pl.program_id(0)

    @pl.when(i == 0)
    def _init():
        m_s[...] = jnp.full((G, 1), -jnp.inf, jnp.float32)
        l_s[...] = jnp.zeros((G, 1), jnp.float32)
        acc_s[...] = jnp.zeros((G, D), jnp.float32)

    x = x_ref[...]
    gwt = gwt_ref[...]
    gate = jax.lax.dot_general(
        gwt, x, (((1,), (1,)), ((), ())),
        preferred_element_type=jnp.float32) + gb_ref[0, 0]
    gate_out_ref[...] = gate[None]
    seg = seg_ref[0]

    sids = jax.lax.broadcasted_iota(jnp.int32, (G, 1), 0)
    onehot = seg == sids

    bm = jnp.max(jnp.where(onehot, gate, -jnp.inf), axis=1, keepdims=True)
    m_old = m_s[...]
    m_new = jnp.maximum(m_old, bm)
    scale = jnp.where(m_new == m_old, 1.0, jnp.exp(m_old - m_new))
    m_row = jnp.sum(jnp.where(onehot, m_new, 0.0), axis=0, keepdims=True)
    p = jnp.exp(gate - m_row)
    w = jnp.where(onehot, p, 0.0)
    l_s[...] = l_s[...] * scale + jnp.sum(w, axis=1, keepdims=True)
    acc_s[...] = acc_s[...] * scale + jax.lax.dot_general(
        w, x, (((1,), (0,)), ((), ())), preferred_element_type=jnp.float32,
        precision=jax.lax.Precision.HIGHEST)
    m_s[...] = m_new

    @pl.when(i == NBLK - 1)
    def _fin():
        l = l_s[...]
        graph_x = acc_s[...] / jnp.where(l > 0, l, 1.0)
        h = jax.lax.dot_general(
            graph_x, w1_ref[...], (((1,), (0,)), ((), ())),
            preferred_element_type=jnp.float32) + b1_ref[...]
        h = jax.nn.gelu(h)
        o = jax.lax.dot_general(
            h, w2_ref[...], (((1,), (0,)), ((), ())),
            preferred_element_type=jnp.float32) + b2_ref[0, 0]
        out_ref[...] = o
        c_out_ref[...] = m_s[...] + jnp.log(jnp.where(l > 0, l, 1.0))


def _sc_body(gate_hbm, seg_hbm, c_hbm, cv_hbm, ci_hbm,
             gate_v, seg_v, c_v, outv_v, outi_v):
    wid = lax.axis_index("s") * 2 + lax.axis_index("c")
    base = jnp.minimum(wid * SHARD, N - SHARD)

    pltpu.sync_copy(c_hbm, c_v)
    pltpu.sync_copy(gate_hbm.at[pl.ds(base, SHARD)], gate_v)
    pltpu.sync_copy(seg_hbm.at[pl.ds(base, SHARD)], seg_v)

    lane = lax.iota(jnp.int32, 16)
    neg = jnp.full((16,), -jnp.inf, jnp.float32)
    zero = jnp.zeros((16,), jnp.int32)
    carry = [neg, zero] * POOL

    cvec = c_v[pl.ds(0, 16)]

    def chunk(j, carry):
        g = gate_v[pl.ds(j * 16, 16)]
        s = seg_v[pl.ds(j * 16, 16)]
        v = g - cvec[s]
        ix = lane + (base + j * 16)
        new = []
        for k in range(POOL):
            tk, uk = carry[2 * k], carry[2 * k + 1]
            m = v > tk
            ntk = jnp.where(m, v, tk)
            nuk = jnp.where(m, ix, uk)
            v = jnp.where(m, tk, v)
            ix = jnp.where(m, uk, ix)
            new.append(ntk)
            new.append(nuk)
        return new

    carry = lax.fori_loop(0, NCHUNK, chunk, carry)
    for k in range(POOL):
        outv_v[pl.ds(k * 16, 16)] = carry[2 * k]
        outi_v[pl.ds(k * 16, 16)] = carry[2 * k + 1]
    pltpu.sync_copy(outv_v, cv_hbm.at[wid])
    pltpu.sync_copy(outi_v, ci_hbm.at[wid])


_sc_topk_candidates = pl.kernel(
    _sc_body,
    mesh=plsc.VectorSubcoreMesh(core_axis_name="c", subcore_axis_name="s"),
    out_type=[
        jax.ShapeDtypeStruct((NW, POOL * 16), jnp.float32),
        jax.ShapeDtypeStruct((NW, POOL * 16), jnp.int32),
    ],
    scratch_types=[
        pltpu.VMEM((SHARD,), jnp.float32),
        pltpu.VMEM((SHARD,), jnp.int32),
        pltpu.VMEM((G,), jnp.float32),
        pltpu.VMEM((POOL * 16,), jnp.float32),
        pltpu.VMEM((POOL * 16,), jnp.int32),
    ],
)


def _merge_body(cv_ref, ci_ref, vals_ref, idx_ref):
    ia = ci_ref[...]
    big = jnp.int32(2**30)

    def body(k, vv):
        mx = jnp.max(vv)
        cand = jnp.where(vv == mx, ia, big)
        ix = jnp.min(cand)
        vals_ref[pl.ds(k, 1), :] = jnp.exp(mx).reshape(1, 1)
        idx_ref[pl.ds(k, 1), :] = ix.reshape(1, 1)
        return jnp.where(ia == ix, -jnp.inf, vv)

    lax.fori_loop(0, TOPK, body, cv_ref[...])


@functools.partial(jax.jit, static_argnames=("interpret",))
def kernel(x, segment_ids, gate_W, gate_b, W1, b1, W2, b2, interpret=False):
    seg3 = segment_ids.reshape(NBLK, 1, BLK)
    gwt = gate_W.reshape(1, D)
    gb = gate_b.reshape(1, 1)
    b1r = b1.reshape(1, D)
    b2r = b2.reshape(1, 1)

    gate_out, out, c = pl.pallas_call(
        _pass1_body,
        grid=(NBLK,),
        in_specs=[
            pl.BlockSpec((1, 1, BLK), lambda i: (i, 0, 0)),
            pl.BlockSpec((BLK, D), lambda i: (i, 0)),
            pl.BlockSpec((1, D), lambda i: (0, 0)),
            pl.BlockSpec((1, 1), lambda i: (0, 0)),
            pl.BlockSpec((D, D), lambda i: (0, 0)),
            pl.BlockSpec((1, D), lambda i: (0, 0)),
            pl.BlockSpec((D, 1), lambda i: (0, 0)),
            pl.BlockSpec((1, 1), lambda i: (0, 0)),
        ],
        out_specs=[
            pl.BlockSpec((1, 1, BLK), lambda i: (i, 0, 0)),
            pl.BlockSpec((G, 1), lambda i: (0, 0)),
            pl.BlockSpec((G, 1), lambda i: (0, 0)),
        ],
        out_shape=[
            jax.ShapeDtypeStruct((NBLK, 1, BLK), jnp.float32),
            jax.ShapeDtypeStruct((G, 1), jnp.float32),
            jax.ShapeDtypeStruct((G, 1), jnp.float32),
        ],
        scratch_shapes=[
            pltpu.VMEM((G, 1), jnp.float32),
            pltpu.VMEM((G, 1), jnp.float32),
            pltpu.VMEM((G, D), jnp.float32),
        ],
        interpret=interpret,
    )(seg3, x, gwt, gb, W1, b1r, W2, b2r)

    cand_v, cand_i = _sc_topk_candidates(
        gate_out.reshape(N), segment_ids, c.reshape(G))

    vals, idx = pl.pallas_call(
        _merge_body,
        out_shape=[
            jax.ShapeDtypeStruct((TOPK, 1), jnp.float32),
            jax.ShapeDtypeStruct((TOPK, 1), jnp.int32),
        ],
        interpret=interpret,
    )(cand_v, cand_i)

    return out, vals.reshape(TOPK), idx.reshape(TOPK)


# speedup vs baseline: 1.3556x; 1.3556x over previous
import functools

import jax
import jax.numpy as jnp
from jax.experimental import pallas as pl
from jax.experimental.pallas import tpu as pltpu

N = 100000
D = 128
G = 16
TOPK = 64
BLK = 10000
NBLK = N // BLK


def _body(seg_ref, x_ref, gwt_ref, gb_ref, w1_ref, b1_ref, w2_ref,
          b2_ref, out_ref, vals_ref, idx_ref,
          m_s, l_s, acc_s, gate_s, seg_s):
    i = pl.program_id(0)

    @pl.when(i == 0)
    def _init():
        m_s[...] = jnp.full((G, 1), -jnp.inf, jnp.float32)
        l_s[...] = jnp.zeros((G, 1), jnp.float32)
        acc_s[...] = jnp.zeros((G, D), jnp.float32)

    x = x_ref[...]
    gwt = gwt_ref[...]
    gate = jax.lax.dot_general(
        gwt, x, (((1,), (1,)), ((), ())),
        preferred_element_type=jnp.float32) + gb_ref[0, 0]
    seg = seg_ref[0]
    gate_s[pl.ds(i, 1), :] = gate
    seg_s[pl.ds(i, 1), :] = seg

    sids = jax.lax.broadcasted_iota(jnp.int32, (G, 1), 0)
    onehot = seg == sids

    bm = jnp.max(jnp.where(onehot, gate, -jnp.inf), axis=1, keepdims=True)
    m_old = m_s[...]
    m_new = jnp.maximum(m_old, bm)
    scale = jnp.where(m_new == m_old, 1.0, jnp.exp(m_old - m_new))
    m_row = jnp.sum(jnp.where(onehot, m_new, 0.0), axis=0, keepdims=True)
    p = jnp.exp(gate - m_row)
    w = jnp.where(onehot, p, 0.0)
    l_s[...] = l_s[...] * scale + jnp.sum(w, axis=1, keepdims=True)
    acc_s[...] = acc_s[...] * scale + jax.lax.dot_general(
        w, x, (((1,), (0,)), ((), ())), preferred_element_type=jnp.float32,
        precision=jax.lax.Precision.HIGHEST)
    m_s[...] = m_new

    @pl.when(i == NBLK - 1)
    def _fin():
        l = l_s[...]
        graph_x = acc_s[...] / jnp.where(l > 0, l, 1.0)
        h = jax.lax.dot_general(
            graph_x, w1_ref[...], (((1,), (0,)), ((), ())),
            preferred_element_type=jnp.float32) + b1_ref[...]
        h = jax.nn.gelu(h)
        o = jax.lax.dot_general(
            h, w2_ref[...], (((1,), (0,)), ((), ())),
            preferred_element_type=jnp.float32) + b2_ref[0, 0]
        out_ref[...] = o

        m = m_s[...]
        c = m + jnp.log(jnp.where(l > 0, l, 1.0))
        g = gate_s[...]
        sg = seg_s[...]
        adj = jnp.full_like(g, -jnp.inf)
        for s in range(G):
            adj = jnp.where(sg == s, g - c[s, 0], adj)

        POOL = 8
        NFULL = BLK // 128
        REM = BLK - NFULL * 128
        pa = [jnp.full((8, 128), -jnp.inf, jnp.float32) for _ in range(POOL)]
        pai = [jnp.zeros((8, 128), jnp.int32) for _ in range(POOL)]
        pb = [jnp.full((2, 128), -jnp.inf, jnp.float32) for _ in range(POOL)]
        pbi = [jnp.zeros((2, 128), jnp.int32) for _ in range(POOL)]
        base8 = (jax.lax.broadcasted_iota(jnp.int32, (8, 128), 0) * BLK
                 + jax.lax.broadcasted_iota(jnp.int32, (8, 128), 1))
        base2 = (jax.lax.broadcasted_iota(jnp.int32, (2, 128), 0) * BLK
                 + jax.lax.broadcasted_iota(jnp.int32, (2, 128), 1))
        for k in range(2):
            rows = 8 if k == 0 else 2
            pv = pa if k == 0 else pb
            pi = pai if k == 0 else pbi
            bb = base8 if k == 0 else base2
            for j in range(NFULL):
                v = jax.lax.slice(adj, (k * 8, j * 128),
                                  (k * 8 + rows, (j + 1) * 128))
                ix = bb + (k * 8 * BLK + j * 128)
                for d in range(POOL):
                    m = v > pv[d]
                    nv = jnp.where(m, v, pv[d])
                    ni = jnp.where(m, ix, pi[d])
                    v = jnp.where(m, pv[d], v)
                    ix = jnp.where(m, pi[d], ix)
                    pv[d] = nv
                    pi[d] = ni
        CA = jax.lax.slice(adj, (0, NFULL * 128), (8, BLK))
        CB = jax.lax.slice(adj, (8, NFULL * 128), (NBLK, BLK))
        CAI = (jax.lax.broadcasted_iota(jnp.int32, (8, REM), 0) * BLK
               + jax.lax.broadcasted_iota(jnp.int32, (8, REM), 1)
               + NFULL * 128)
        CBI = (jax.lax.broadcasted_iota(jnp.int32, (2, REM), 0) * BLK
               + jax.lax.broadcasted_iota(jnp.int32, (2, REM), 1)
               + (8 * BLK + NFULL * 128))
        PA = jnp.concatenate(pa, axis=0)
        PAI = jnp.concatenate(pai, axis=0)
        PB = jnp.concatenate(pb, axis=0)
        PBI = jnp.concatenate(pbi, axis=0)

        big = jnp.int32(2**30)

        def body(kk, carry):
            A, B, Ca, Cb = carry
            mx = jnp.maximum(jnp.maximum(jnp.max(A), jnp.max(B)),
                             jnp.maximum(jnp.max(Ca), jnp.max(Cb)))
            ix = jnp.minimum(
                jnp.minimum(jnp.min(jnp.where(A == mx, PAI, big)),
                            jnp.min(jnp.where(B == mx, PBI, big))),
                jnp.minimum(jnp.min(jnp.where(Ca == mx, CAI, big)),
                            jnp.min(jnp.where(Cb == mx, CBI, big))))
            vals_ref[pl.ds(kk, 1), :] = jnp.exp(mx).reshape(1, 1)
            idx_ref[pl.ds(kk, 1), :] = ix.reshape(1, 1)
            return (jnp.where(PAI == ix, -jnp.inf, A),
                    jnp.where(PBI == ix, -jnp.inf, B),
                    jnp.where(CAI == ix, -jnp.inf, Ca),
                    jnp.where(CBI == ix, -jnp.inf, Cb))

        jax.lax.fori_loop(0, TOPK, body, (PA, PB, CA, CB))


@functools.partial(jax.jit, static_argnames=("interpret",))
def kernel(x, segment_ids, gate_W, gate_b, W1, b1, W2, b2, interpret=False):
    seg3 = segment_ids.reshape(NBLK, 1, BLK)
    gwt = gate_W.reshape(1, D)
    gb = gate_b.reshape(1, 1)
    b1r = b1.reshape(1, D)
    b2r = b2.reshape(1, 1)

    out, vals, idx = pl.pallas_call(
        _body,
        grid=(NBLK,),
        in_specs=[
            pl.BlockSpec((1, 1, BLK), lambda i: (i, 0, 0)),
            pl.BlockSpec((BLK, D), lambda i: (i, 0)),
            pl.BlockSpec((1, D), lambda i: (0, 0)),
            pl.BlockSpec((1, 1), lambda i: (0, 0)),
            pl.BlockSpec((D, D), lambda i: (0, 0)),
            pl.BlockSpec((1, D), lambda i: (0, 0)),
            pl.BlockSpec((D, 1), lambda i: (0, 0)),
            pl.BlockSpec((1, 1), lambda i: (0, 0)),
        ],
        out_specs=[
            pl.BlockSpec((G, 1), lambda i: (0, 0)),
            pl.BlockSpec((TOPK, 1), lambda i: (0, 0)),
            pl.BlockSpec((TOPK, 1), lambda i: (0, 0)),
        ],
        out_shape=[
            jax.ShapeDtypeStruct((G, 1), jnp.float32),
            jax.ShapeDtypeStruct((TOPK, 1), jnp.float32),
            jax.ShapeDtypeStruct((TOPK, 1), jnp.int32),
        ],
        scratch_shapes=[
            pltpu.VMEM((G, 1), jnp.float32),
            pltpu.VMEM((G, 1), jnp.float32),
            pltpu.VMEM((G, D), jnp.float32),
            pltpu.VMEM((NBLK, BLK), jnp.float32),
            pltpu.VMEM((NBLK, BLK), jnp.int32),
        ],
        interpret=interpret,
    )(seg3, x, gwt, gb, W1, b1r, W2, b2r)

    return out, vals.reshape(TOPK), idx.reshape(TOPK)
